# baseline (speedup 1.0000x reference)
import functools

import jax
import jax.numpy as jnp
from jax import lax
from jax.experimental import pallas as pl
from jax.experimental.pallas import tpu as pltpu
from jax.experimental.pallas import tpu_sc as plsc

B = 16384
C = 1000
EPS = 1e-8

L = 16
NROWS = (B * C) // L
K = 24
G = 4
NC = 2
NS = 16
NW = NC * NS
BPW = B // NW
SLOTS = BPW * G
CH = 128
NCH = SLOTS // CH
NI = BPW // L


def _sc_body(x_hbm, y_hbm, out_hbm, y_v, idx_v, rows_v, s_v, sem):
    wid = lax.axis_index("s") * NC + lax.axis_index("c")
    base = wid * BPW
    pltpu.sync_copy(y_hbm.at[pl.ds(base, BPW)], y_v)
    iota = lax.iota(jnp.int32, L)

    def build(i, carry):
        y_vec = y_v[pl.ds(i * L, L)]
        b_vec = (base + i * L) + iota
        flat = b_vec * C + y_vec - K
        n0 = lax.shift_right_arithmetic(flat, 4)
        for g in range(G):
            idx_v[pl.ds(g * BPW + i * L, L)] = jnp.clip(n0 + g, 0, NROWS - 1)
        return carry

    lax.fori_loop(0, NI, build, 0)

    copies = [
        pltpu.async_copy(
            x_hbm.at[idx_v.at[pl.ds(c * CH, CH)]],
            rows_v.at[pl.ds(c * CH, CH)],
            sem,
        )
        for c in range(NCH)
    ]
    for cp in copies:
        cp.wait()

    lane_ids = [jnp.full((L,), lane, jnp.int32) for lane in range(L)]

    def compute(i, carry):
        y_vec = y_v[pl.ds(i * L, L)]
        bl_vec = i * L + iota
        flat = (base + i * L + iota) * C + y_vec - K
        dbase = lax.bitwise_and(flat, 15) + K
        acc = jnp.zeros((L,), jnp.float32)
        for g in range(G):
            row_g = bl_vec + g * BPW
            for lane in range(L):
                jj = g * L + lane
                v = plsc.load_gather(rows_v, [row_g, lane_ids[lane]])
                d = jj - dbase
                col = y_vec + d
                valid = (col >= 0) & (col < C)
                w = lax.bitcast_convert_type(
                    lax.shift_left(127 - jnp.abs(d), 23), jnp.float32
                )
                w = jnp.where(valid, w, 0.0)
                acc = acc + w * v
        s_v[pl.ds(i * L, L)] = acc
        return carry

    lax.fori_loop(0, NI, compute, 0)
    pltpu.sync_copy(s_v, out_hbm.at[pl.ds(base, BPW)])


_sc_call = functools.partial(
    pl.kernel,
    out_type=jax.ShapeDtypeStruct((B,), jnp.float32),
    mesh=plsc.VectorSubcoreMesh(core_axis_name="c", subcore_axis_name="s"),
    scratch_types=[
        pltpu.VMEM((BPW,), jnp.int32),
        pltpu.VMEM((SLOTS,), jnp.int32),
        pltpu.VMEM((SLOTS, L), jnp.float32),
        pltpu.VMEM((BPW,), jnp.float32),
        pltpu.SemaphoreType.DMA,
    ],
    compiler_params=pltpu.CompilerParams(
        needs_layout_passes=False, use_tc_tiling_on_sc=False
    ),
)(_sc_body)


def _loss_body(s_ref, o_ref):
    t = -jnp.log(s_ref[...] + EPS)
    o_ref[0, 0] = jnp.sum(jnp.sum(t, axis=1)) * (1.0 / B)


_loss_call = pl.pallas_call(
    _loss_body,
    out_shape=jax.ShapeDtypeStruct((1, 1), jnp.float32),
    in_specs=[pl.BlockSpec(memory_space=pltpu.VMEM)],
    out_specs=pl.BlockSpec(memory_space=pltpu.SMEM),
)


def kernel(x, y):
    s = _sc_call(x.reshape(NROWS, L), y.astype(jnp.int32))
    return _loss_call(s.reshape(128, 128))[0, 0]


# speedup vs baseline: 3.7022x; 3.7022x over previous
import jax
import jax.numpy as jnp
from jax import lax
from jax.experimental import pallas as pl
from jax.experimental.pallas import tpu as pltpu

B = 16384
C = 1000
EPS = 1e-8
BB = 512
NB = B // BB


def _body(y_ref, xt_ref, o_ref):
    j = pl.program_id(0)
    xblk = xt_ref[...]
    yv = y_ref[0]
    c = lax.broadcasted_iota(jnp.int32, (C, BB), 0)
    ad = jnp.minimum(jnp.abs(c - yv), 127)
    w = lax.bitcast_convert_type(lax.shift_left(127 - ad, 23), jnp.float32)
    s = jnp.sum(w * xblk, axis=0)
    part = jnp.sum(-jnp.log(s + EPS)) * (1.0 / B)

    @pl.when(j == 0)
    def _():
        o_ref[0, 0] = 0.0

    o_ref[0, 0] += part


_call = pl.pallas_call(
    _body,
    grid=(NB,),
    in_specs=[
        pl.BlockSpec((1, 1, BB), lambda j: (j, 0, 0)),
        pl.BlockSpec((C, BB), lambda j: (0, j)),
    ],
    out_specs=pl.BlockSpec((1, 1), lambda j: (0, 0), memory_space=pltpu.SMEM),
    out_shape=jax.ShapeDtypeStruct((1, 1), jnp.float32),
)


def kernel(x, y):
    y3 = y.astype(jnp.int32).reshape(NB, 1, BB)
    return _call(y3, x.T)[0, 0]


# speedup vs baseline: 5.1379x; 1.3878x over previous
import jax
import jax.numpy as jnp
import numpy as np
from jax import lax
from jax.experimental import pallas as pl
from jax.experimental.pallas import tpu as pltpu

B = 16384
C = 1000
EPS = 1e-8
CB = 40
NB = C // CB
SUB = 8
SIGN = np.int32(-2147483648)


def _w(m, base):
    df = m + lax.convert_element_type(base, jnp.float32)
    na = lax.bitcast_convert_type(
        lax.bitcast_convert_type(df, jnp.int32) | SIGN, jnp.float32
    )
    return jnp.exp2(na)


def _body(y_ref, xt_ref, o_ref, acc_ref, m_ref):
    j = pl.program_id(0)

    @pl.when(j == 0)
    def _():
        iota = lax.broadcasted_iota(jnp.int32, (SUB, B), 0)
        m_ref[...] = (iota - y_ref[...]).astype(jnp.float32)
        acc_ref[...] = jnp.zeros_like(acc_ref)

    m = m_ref[...]
    acc_ref[...] += sum(
        _w(m, j * CB + k * SUB) * xt_ref[pl.ds(k * SUB, SUB), :]
        for k in range(CB // SUB)
    )

    @pl.when(j == NB - 1)
    def _():
        s = jnp.sum(acc_ref[...], axis=0, keepdims=True)
        o_ref[0, 0] = jnp.sum(-jnp.log(s + EPS)) * (1.0 / B)


_call = pl.pallas_call(
    _body,
    grid=(NB,),
    in_specs=[
        pl.BlockSpec((1, B), lambda j: (0, 0)),
        pl.BlockSpec((CB, B), lambda j: (j, 0)),
    ],
    out_specs=pl.BlockSpec((1, 1), lambda j: (0, 0), memory_space=pltpu.SMEM),
    out_shape=jax.ShapeDtypeStruct((1, 1), jnp.float32),
    scratch_shapes=[
        pltpu.VMEM((SUB, B), jnp.float32),
        pltpu.VMEM((SUB, B), jnp.float32),
    ],
)


def kernel(x, y):
    y2 = y.astype(jnp.int32).reshape(1, B)
    return _call(y2, x.T)[0, 0]
